# baseline (speedup 1.0000x reference)
import functools

import jax
import jax.numpy as jnp
from jax import lax
from jax.experimental import pallas as pl
from jax.experimental.pallas import tpu as pltpu
from jax.experimental.pallas import tpu_sc as plsc

EPS = 1e-6
NS = 0.2
B = 8
N = 2048
K = 20
BLK = 256
NB = N // BLK
NEG = -3.0e38


def _dot(a, b):
    return jnp.dot(a, b, precision=lax.Precision.HIGHEST)


def _rt(a):
    return a.astype(jnp.bfloat16).astype(jnp.float32)


def _bdot(a, b):
    return jnp.dot(_rt(a), b.astype(jnp.float32),
                   precision=lax.Precision.HIGHEST)


def _knn_body(fa_ref, fb_ref, xc_ref, xq_ref, idx_ref):
    fa = fa_ref[0]
    fb = fb_ref[0]
    inner = lax.dot_general(_rt(fa), _rt(fb), (((1,), (1,)), ((), ())),
                            precision=lax.Precision.HIGHEST)
    xa = xc_ref[0]
    xb = xq_ref[0]
    pd = (2.0 * inner - xb) - xa
    sub = lax.broadcasted_iota(jnp.int32, (N, BLK), 0)
    base = pl.program_id(0) * N
    rows = []
    for _ in range(K):
        m = jnp.max(pd, axis=0, keepdims=True)
        pref = jnp.where(pd == m, N - sub, 0)
        am = N - jnp.max(pref, axis=0, keepdims=True)
        rows.append(am + base)
        pd = jnp.where(sub == am, NEG, pd)
    idx_ref[:, 0, 0, :] = jnp.concatenate(rows, axis=0)


def _knn(f, fch):
    d = f.shape[-1]
    xx = jnp.sum(fch * fch, axis=1)
    return pl.pallas_call(
        _knn_body,
        grid=(B, NB),
        in_specs=[
            pl.BlockSpec((1, N, d), lambda b, i: (b, 0, 0)),
            pl.BlockSpec((1, BLK, d), lambda b, i: (b, i, 0)),
            pl.BlockSpec((1, N, 1), lambda b, i: (b, 0, 0)),
            pl.BlockSpec((1, 1, BLK), lambda b, i: (b, 0, i)),
        ],
        out_specs=pl.BlockSpec((K, 1, 1, BLK), lambda b, i: (0, b, 0, i)),
        out_shape=jax.ShapeDtypeStruct((K, B, 1, N), jnp.int32),
        compiler_params=pltpu.CompilerParams(
            dimension_semantics=("arbitrary", "arbitrary")),
    )(f, f, xx[:, :, None], xx[:, None, :])


def _sc_gather(table, idx_flat):
    rdim = table.shape[1]
    m = idx_flat.shape[0]
    n_cores, n_sub = 2, 16
    b_per_w = m // (n_cores * n_sub)
    ch = 512
    n_ch = b_per_w // ch
    mesh = plsc.VectorSubcoreMesh(core_axis_name="c", subcore_axis_name="s")

    @functools.partial(
        pl.kernel, mesh=mesh,
        out_type=jax.ShapeDtypeStruct((m, rdim), jnp.float32),
        scratch_types=[
            pltpu.VMEM((ch,), jnp.int32),
            pltpu.VMEM((ch, rdim), jnp.float32),
            pltpu.SemaphoreType.DMA,
        ],
        compiler_params=pltpu.CompilerParams(use_tc_tiling_on_sc=False),
    )
    def gk(table_hbm, idx_hbm, out_hbm, idx_v, rows_v, sem):
        wid = lax.axis_index("s") * n_cores + lax.axis_index("c")
        base = wid * b_per_w

        def body(i, carry):
            off = base + i * ch
            pltpu.sync_copy(idx_hbm.at[pl.ds(off, ch)], idx_v)
            pltpu.async_copy(table_hbm.at[idx_v], rows_v, sem).wait()
            pltpu.sync_copy(rows_v, out_hbm.at[pl.ds(off, ch)])
            return carry

        lax.fori_loop(0, n_ch, body, 0)

    return gk(table, idx_flat)


def _vn_norm(p, s_mat):
    return jnp.sqrt(_dot(p * p, s_mat)) + EPS


def _vn_apply(p, d, norm, mean, var, s_mat, t_mat, dns=None):
    norm_bn = (norm - mean) / jnp.sqrt(var + 1e-5)
    p = p / _dot(norm, t_mat) * _dot(norm_bn, t_mat)
    dot = _dot(p * d, s_mat)
    mask = _dot((dot >= 0).astype(jnp.float32), t_mat)
    if dns is None:
        dns = _dot(d * d, s_mat)
    ratio = _dot(dot / (dns + EPS), t_mat)
    return NS * p + (1.0 - NS) * (mask * p + (1.0 - mask) * (p - ratio * d))


def _stage(gath, f, wg, wh, wdg, wdh, w2g, w2d, s_mat, t_mat, two_layer):
    d = f.shape[-1]
    npass = 5 if two_layer else 3
    m_edges = float(K * B * N)

    def body(g_ref, f_ref, wg_ref, wh_ref, wdg_ref, wdh_ref,
             w2g_ref, w2d_ref, s_ref, t_ref, out_ref, acc):
        p_id = pl.program_id(0)
        b_id = pl.program_id(1)
        i_id = pl.program_id(2)
        first = jnp.logical_and(p_id == 0,
                                jnp.logical_and(b_id == 0, i_id == 0))

        @pl.when(first)
        def _():
            acc[...] = jnp.zeros_like(acc)

        s_m = s_ref[...]
        t_m = t_ref[...]
        xj = g_ref[:, 0]
        xn = f_ref[0]
        diff = (xj - xn[None]).reshape(K * BLK, d)

        def edges():
            p3 = (_bdot(diff, wg_ref[...]).reshape(K, BLK, 64)
                  + _bdot(xn, wh_ref[...])[None]).reshape(K * BLK, 64)
            d3 = (_bdot(diff, wdg_ref[...]).reshape(K, BLK, 64)
                  + _bdot(xn, wdh_ref[...])[None]).reshape(K * BLK, 64)
            return p3, d3

        def layer1():
            p1, d1 = edges()
            norm1 = _vn_norm(p1, s_m)
            mean1 = acc[0:1, :] / m_edges
            var1 = acc[1:2, :] / m_edges
            return _vn_apply(p1, d1, norm1, mean1, var1, s_m, t_m)

        @pl.when(p_id == 0)
        def _():
            p1, _d1 = edges()
            norm1 = _vn_norm(p1, s_m)
            acc[0:1, :] += jnp.sum(norm1, axis=0, keepdims=True)

        @pl.when(p_id == 1)
        def _():
            p1, _d1 = edges()
            norm1 = _vn_norm(p1, s_m)
            c = norm1 - acc[0:1, :] / m_edges
            acc[1:2, :] += jnp.sum(c * c, axis=0, keepdims=True)

        if two_layer:
            @pl.when(p_id == 2)
            def _():
                h1 = layer1()
                norm2 = _vn_norm(_bdot(h1, w2g_ref[...]), s_m)
                acc[2:3, :] += jnp.sum(norm2, axis=0, keepdims=True)

            @pl.when(p_id == 3)
            def _():
                h1 = layer1()
                norm2 = _vn_norm(_bdot(h1, w2g_ref[...]), s_m)
                c = norm2 - acc[2:3, :] / m_edges
                acc[3:4, :] += jnp.sum(c * c, axis=0, keepdims=True)

            @pl.when(p_id == 4)
            def _():
                h1 = layer1()
                p2 = _bdot(h1, w2g_ref[...])
                d2 = _bdot(h1, w2d_ref[...])
                norm2 = _vn_norm(p2, s_m)
                mean2 = acc[2:3, :] / m_edges
                var2 = acc[3:4, :] / m_edges
                h2 = _vn_apply(p2, d2, norm2, mean2, var2, s_m, t_m)
                out_ref[0] = jnp.mean(h2.reshape(K, BLK, 64), axis=0)
        else:
            @pl.when(p_id == 2)
            def _():
                h1 = layer1()
                out_ref[0] = jnp.mean(h1.reshape(K, BLK, 64), axis=0)

    return pl.pallas_call(
        body,
        grid=(npass, B, NB),
        in_specs=[
            pl.BlockSpec((K, 1, BLK, d), lambda p, b, i: (0, b, i, 0)),
            pl.BlockSpec((1, BLK, d), lambda p, b, i: (b, i, 0)),
            pl.BlockSpec((d, 64), lambda p, b, i: (0, 0)),
            pl.BlockSpec((d, 64), lambda p, b, i: (0, 0)),
            pl.BlockSpec((d, 64), lambda p, b, i: (0, 0)),
            pl.BlockSpec((d, 64), lambda p, b, i: (0, 0)),
            pl.BlockSpec((64, 64), lambda p, b, i: (0, 0)),
            pl.BlockSpec((64, 64), lambda p, b, i: (0, 0)),
            pl.BlockSpec((64, 64), lambda p, b, i: (0, 0)),
            pl.BlockSpec((64, 64), lambda p, b, i: (0, 0)),
        ],
        out_specs=pl.BlockSpec(
            (1, BLK, 64),
            lambda p, b, i: (jnp.where(p == npass - 1, b, B),
                             jnp.where(p == npass - 1, i, 0), 0)),
        out_shape=jax.ShapeDtypeStruct((B + 1, N, 64), jnp.float32),
        scratch_shapes=[pltpu.VMEM((8, 64), jnp.float32)],
        compiler_params=pltpu.CompilerParams(
            dimension_semantics=("arbitrary", "arbitrary", "arbitrary")),
    )(gath, f, wg, wh, wdg, wdh, w2g, w2d, s_mat, t_mat)[:B]


def _final(x123, w6g, w6dg, wdx, s6, t6):
    m_pts = float(B * N)

    def body(x_ref, wg_ref, wd_ref, wdx_ref, s_ref, t_ref, out_ref, acc):
        p_id = pl.program_id(0)
        b_id = pl.program_id(1)
        i_id = pl.program_id(2)
        first = jnp.logical_and(p_id == 0,
                                jnp.logical_and(b_id == 0, i_id == 0))

        @pl.when(first)
        def _():
            acc[...] = jnp.zeros_like(acc)

        xb = x_ref[0]
        p = _bdot(xb, wg_ref[...])
        norm = _vn_norm(p, s_ref[...])

        @pl.when(p_id == 0)
        def _():
            acc[0:1, :512] += jnp.sum(norm, axis=0, keepdims=True)

        @pl.when(p_id == 1)
        def _():
            c = norm - acc[0:1, :512] / m_pts
            acc[1:2, :512] += jnp.sum(c * c, axis=0, keepdims=True)

        @pl.when(p_id == 2)
        def _():
            mean = acc[0:1, :512] / m_pts
            var = acc[1:2, :512] / m_pts
            dv = _bdot(xb, wd_ref[...])
            dexp = _dot(dv, wdx_ref[...])
            dns = jnp.sum(dv * dv, axis=1, keepdims=True)
            h = _vn_apply(p, dexp, norm, mean, var, s_ref[...], t_ref[...],
                          dns=dns)

            @pl.when(i_id == 0)
            def _():
                acc[2:3, :] = jnp.zeros_like(acc[2:3])

            acc[2:3, :] += jnp.sum(h, axis=0, keepdims=True)

            @pl.when(i_id == NB - 1)
            def _():
                out_ref[0] = acc[2:3, :] / N

    return pl.pallas_call(
        body,
        grid=(3, B, NB),
        in_specs=[
            pl.BlockSpec((1, BLK, 192), lambda p, b, i: (b, i, 0)),
            pl.BlockSpec((192, 1536), lambda p, b, i: (0, 0)),
            pl.BlockSpec((192, 16), lambda p, b, i: (0, 0)),
            pl.BlockSpec((16, 1536), lambda p, b, i: (0, 0)),
            pl.BlockSpec((1536, 512), lambda p, b, i: (0, 0)),
            pl.BlockSpec((512, 1536), lambda p, b, i: (0, 0)),
        ],
        out_specs=pl.BlockSpec(
            (1, 1, 1536), lambda p, b, i: (jnp.where(p == 2, b, B), 0, 0)),
        out_shape=jax.ShapeDtypeStruct((B + 1, 1, 1536), jnp.float32),
        scratch_shapes=[pltpu.VMEM((8, 1536), jnp.float32)],
        compiler_params=pltpu.CompilerParams(
            dimension_semantics=("arbitrary", "arbitrary", "arbitrary")),
    )(x123, w6g, w6dg, wdx, s6, t6)[:B]


def _expand64(w):
    m = jnp.kron(w.T.astype(jnp.bfloat16), jnp.eye(3, dtype=jnp.bfloat16))
    return jnp.pad(m, ((0, 1), (0, 1)))


def _expand_vec(v, rows):
    m = jnp.kron(v[None, :].astype(jnp.bfloat16),
                 jnp.eye(3, dtype=jnp.bfloat16))
    return jnp.pad(m, ((0, rows - 3), (0, 1)))


def _sel_mats(nch):
    r = jnp.arange(nch * 3)
    s = jnp.zeros((nch * 3, nch), jnp.float32).at[r, r // 3].set(1.0)
    return s, s.T


def kernel(x, W1f, W1d, W2f, W2d, W3f, W3d, W4f, W4d, W5f, W5d, W6f, W6d):
    s_raw, t_raw = _sel_mats(21)
    s_mat = jnp.pad(s_raw, ((0, 1), (0, 43)))
    t_mat = jnp.pad(t_raw, ((0, 43), (0, 1))).at[63, 63].set(1.0)

    pts = jnp.pad(jnp.transpose(x, (0, 2, 1)), ((0, 0), (0, 0), (0, 13)))
    w1g = _expand_vec(W1f[:, 0], 16)
    w1h = _expand_vec(W1f[:, 1], 16)
    w1dg = _expand_vec(W1d[:, 0], 16)
    w1dh = _expand_vec(W1d[:, 1], 16)
    idx1 = _knn(pts, x).reshape(-1)
    g1 = _sc_gather(pts.reshape(B * N, 16), idx1).reshape(K, B, N, 16)
    x1 = _stage(g1, pts, w1g, w1h, w1dg, w1dh,
                _expand64(W2f), _expand64(W2d), s_mat, t_mat, True)

    idx2 = _knn(x1, jnp.transpose(x1, (0, 2, 1))[:, :63, :]).reshape(-1)
    g2 = _sc_gather(x1.reshape(B * N, 64), idx2).reshape(K, B, N, 64)
    x2 = _stage(g2, x1, _expand64(W3f[:, :21]), _expand64(W3f[:, 21:]),
                _expand64(W3d[:, :21]), _expand64(W3d[:, 21:]),
                _expand64(W4f), _expand64(W4d), s_mat, t_mat, True)

    idx3 = _knn(x2, jnp.transpose(x2, (0, 2, 1))[:, :63, :]).reshape(-1)
    g3 = _sc_gather(x2.reshape(B * N, 64), idx3).reshape(K, B, N, 64)
    x3 = _stage(g3, x2, _expand64(W5f[:, :21]), _expand64(W5f[:, 21:]),
                _expand64(W5d[:, :21]), _expand64(W5d[:, 21:]),
                _expand64(W4f), _expand64(W4d), s_mat, t_mat, False)

    x123 = jnp.concatenate([x1, x2, x3], axis=2)
    s6, t6 = _sel_mats(512)
    eye3b = jnp.eye(3, dtype=jnp.bfloat16)
    w6g = jnp.concatenate(
        [jnp.pad(jnp.kron(W6f[:, 21 * s:21 * (s + 1)].T.astype(jnp.bfloat16),
                          eye3b), ((0, 1), (0, 0)))
         for s in range(3)], axis=0)
    w6dg = jnp.concatenate(
        [jnp.pad(jnp.kron(W6d[:, 21 * s:21 * (s + 1)].T.astype(jnp.bfloat16),
                          eye3b), ((0, 1), (0, 13)))
         for s in range(3)], axis=0)
    wdx = jnp.pad(jnp.tile(jnp.eye(3, dtype=jnp.float32), (1, 512)),
                  ((0, 13), (0, 0)))
    out = _final(x123, w6g, w6dg, wdx, s6, t6)
    return out.reshape(B, 512, 3)
